# baseline (speedup 1.0000x reference)
import functools

import jax
import jax.numpy as jnp
from jax.experimental import pallas as pl
from jax.experimental.pallas import tpu as pltpu

N_NODES = 10000
N_EDGES = 320000
H = 128
L = 3


def _colsum_body(x_ref, o_ref):
    @pl.when(pl.program_id(0) == 0)
    def _init():
        o_ref[...] = jnp.zeros_like(o_ref)

    o_ref[...] += jnp.sum(x_ref[...], axis=0, keepdims=True)


def _colsum(x, block_rows):
    n = x.shape[0]
    assert n % block_rows == 0
    return pl.pallas_call(
        _colsum_body,
        grid=(n // block_rows,),
        in_specs=[pl.BlockSpec((block_rows, H), lambda i: (i, 0))],
        out_specs=pl.BlockSpec((1, H), lambda i: (0, 0)),
        out_shape=jax.ShapeDtypeStruct((1, H), jnp.float32),
    )(x)


def _readout_body(hs_ref, es_ref, w1_ref, b1_ref, w2_ref, b2_ref, o_ref):
    x = jnp.concatenate([hs_ref[...] / N_NODES, es_ref[...] / N_EDGES], axis=-1)
    y = x @ w1_ref[...] + b1_ref[...]
    y = y * jax.nn.sigmoid(y)
    o_ref[...] = y @ w2_ref[...] + b2_ref[...]


def _readout(hsum, esum, w1, b1, w2, b2):
    return pl.pallas_call(
        _readout_body,
        out_shape=jax.ShapeDtypeStruct((1, 1), jnp.float32),
    )(hsum, esum, w1, b1.reshape(1, H), w2, b2.reshape(1, 1))


def _graph_conv(h, src, dst, n, W, b):
    out_deg = jnp.clip(jnp.zeros((n,), h.dtype).at[src].add(1.0), 1.0, None)
    h = h * (out_deg ** -0.5)[:, None]
    msg = jnp.take(h, src, axis=0)
    agg = jnp.zeros((n, h.shape[1]), h.dtype).at[dst].add(msg)
    in_deg = jnp.clip(jnp.zeros((n,), h.dtype).at[dst].add(1.0), 1.0, None)
    agg = agg * (in_deg ** -0.5)[:, None]
    return agg @ W + b


def kernel(z, g_edge_index, d, lg_edge_index, emb_table, edge_proj_w, edge_proj_b,
           g_w, g_b, lg_w, lg_b, r_w1, r_b1, r_w2, r_b2):
    h = jnp.take(emb_table, z, axis=0)
    e = d @ edge_proj_w + edge_proj_b
    g_src, g_dst = g_edge_index[0], g_edge_index[1]
    lg_src, lg_dst = lg_edge_index[0], lg_edge_index[1]
    for i in range(L):
        h = jax.nn.relu(_graph_conv(h, g_src, g_dst, N_NODES, g_w[i], g_b[i]))
    for i in range(L):
        e = jax.nn.relu(_graph_conv(e, lg_src, lg_dst, N_EDGES, lg_w[i], lg_b[i]))
    hsum = _colsum(h, 2000)
    esum = _colsum(e, 1000)
    out = _readout(hsum, esum, r_w1, r_b1, r_w2, r_b2)
    return out.reshape(-1)


# speedup vs baseline: 1.2214x; 1.2214x over previous
import functools

import jax
import jax.numpy as jnp
from jax import lax
from jax.experimental import pallas as pl
from jax.experimental.pallas import tpu as pltpu
from jax.experimental.pallas import tpu_sc as plsc

N_NODES = 10000
N_EDGES = 320000
LG_EDGES = 640000
H = 128
L = 3

_NC, _NS = 2, 16
_PH = jax.lax.Precision.HIGHEST

_sc_mesh = plsc.VectorSubcoreMesh(core_axis_name="c", subcore_axis_name="s")


def _prep_body(gei, lgei, z, emb, degp_g, degp_lg, h0,
               zbuf, ones, idxb, zidx, rows, obl, ibl, obg, ibg):
    c = lax.axis_index("c")
    s = lax.axis_index("s")
    w = c * _NS + s

    def zf(i, carry):
        zbuf[pl.ds(i * 16, 16)] = jnp.zeros((16,), jnp.float32)
        return carry

    lax.fori_loop(0, 250, zf, 0)
    for i in range(5):
        ones[pl.ds(i * 16, 16)] = jnp.ones((16,), jnp.float32)

    for r in (obl, ibl):
        for k in range(5):
            pltpu.sync_copy(zbuf, r.at[pl.ds(s * 20000 + k * 4000, 4000)])

    @pl.when(s < 10)
    def _zg():
        pltpu.sync_copy(zbuf.at[pl.ds(0, 1000)], obg.at[pl.ds(s * 1000, 1000)])
        pltpu.sync_copy(zbuf.at[pl.ds(0, 1000)], ibg.at[pl.ds(s * 1000, 1000)])

    plsc.subcore_barrier()

    base_lg = c * 320000 + s * 20000

    def dlg(t, carry):
        off = base_lg + t * 80
        pltpu.sync_copy(lgei.at[pl.ds(off, 80)], idxb.at[0])
        pltpu.sync_copy(lgei.at[pl.ds(LG_EDGES + off, 80)], idxb.at[1])
        pltpu.sync_copy(ones, obl.at[idxb.at[0]], add=True)
        pltpu.sync_copy(ones, ibl.at[idxb.at[1]], add=True)
        return carry

    lax.fori_loop(0, 250, dlg, 0)

    base_g = c * 160000 + s * 10000

    def dg(t, carry):
        off = base_g + t * 80
        pltpu.sync_copy(gei.at[pl.ds(off, 80)], idxb.at[0])
        pltpu.sync_copy(gei.at[pl.ds(N_EDGES + off, 80)], idxb.at[1])
        pltpu.sync_copy(ones, obg.at[idxb.at[0]], add=True)
        pltpu.sync_copy(ones, ibg.at[idxb.at[1]], add=True)
        return carry

    lax.fori_loop(0, 125, dg, 0)

    def egather(k, carry):
        t = w + 32 * k

        @pl.when(t < 125)
        def _eg():
            off = t * 80
            pltpu.sync_copy(z.at[pl.ds(off, 80)], zidx.at[0])
            pltpu.sync_copy(emb.at[zidx.at[0]], rows)
            pltpu.sync_copy(rows, h0.at[pl.ds(off, 80)])

        return carry

    lax.fori_loop(0, 4, egather, 0)

    plsc.subcore_barrier()

    for j, r in ((0, obl), (1, ibl)):
        for k in range(5):
            pltpu.sync_copy(r.at[pl.ds(s * 20000 + k * 4000, 4000)], zbuf)
            pltpu.sync_copy(zbuf, degp_lg.at[pl.ds((2 * j + c) * N_EDGES + s * 20000 + k * 4000, 4000)])

    @pl.when(s < 10)
    def _wg():
        for j, r in ((0, obg), (1, ibg)):
            pltpu.sync_copy(r.at[pl.ds(s * 1000, 1000)], zbuf.at[pl.ds(0, 1000)])
            pltpu.sync_copy(zbuf.at[pl.ds(0, 1000)], degp_g.at[pl.ds((2 * j + c) * N_NODES + s * 1000, 1000)])


_prep = pl.kernel(
    _prep_body,
    out_type=(
        jax.ShapeDtypeStruct((4 * N_NODES,), jnp.float32),
        jax.ShapeDtypeStruct((4 * N_EDGES,), jnp.float32),
        jax.ShapeDtypeStruct((N_NODES, H), jnp.float32),
    ),
    mesh=_sc_mesh,
    scratch_types=[
        pltpu.VMEM((4000,), jnp.float32),
        pltpu.VMEM((80,), jnp.float32),
        pltpu.VMEM((2, 80), jnp.int32),
        pltpu.VMEM((1, 80), jnp.int32),
        pltpu.VMEM((80, H), jnp.float32),
        pltpu.VMEM_SHARED((N_EDGES,), jnp.float32),
        pltpu.VMEM_SHARED((N_EDGES,), jnp.float32),
        pltpu.VMEM_SHARED((N_NODES,), jnp.float32),
        pltpu.VMEM_SHARED((N_NODES,), jnp.float32),
    ],
)


def _colsum_body(x_ref, o_ref):
    @pl.when(pl.program_id(0) == 0)
    def _init():
        o_ref[...] = jnp.zeros_like(o_ref)

    o_ref[...] += jnp.sum(x_ref[...], axis=0, keepdims=True)


def _colsum(x, block_rows):
    n = x.shape[0]
    return pl.pallas_call(
        _colsum_body,
        grid=(n // block_rows,),
        in_specs=[pl.BlockSpec((block_rows, H), lambda i: (i, 0))],
        out_specs=pl.BlockSpec((1, H), lambda i: (0, 0)),
        out_shape=jax.ShapeDtypeStruct((1, H), jnp.float32),
    )(x)


def _readout_body(hs_ref, es_ref, w1_ref, b1_ref, w2_ref, b2_ref, o_ref):
    x = jnp.concatenate([hs_ref[...] / N_NODES, es_ref[...] / N_EDGES], axis=-1)
    y = jnp.dot(x, w1_ref[...], precision=_PH) + b1_ref[...]
    y = y * jax.nn.sigmoid(y)
    o_ref[...] = jnp.dot(y, w2_ref[...], precision=_PH) + b2_ref[...]


def _readout(hsum, esum, w1, b1, w2, b2):
    return pl.pallas_call(
        _readout_body,
        out_shape=jax.ShapeDtypeStruct((1, 1), jnp.float32),
    )(hsum, esum, w1, b1.reshape(1, H), w2, b2.reshape(1, 1))


def _graph_conv(h, src, dst, n, rs_out, rs_in, W, b):
    h = h * rs_out[:, None]
    msg = jnp.take(h, src, axis=0)
    agg = jnp.zeros((n, h.shape[1]), h.dtype).at[dst].add(msg)
    agg = agg * rs_in[:, None]
    return agg @ W + b


def kernel(z, g_edge_index, d, lg_edge_index, emb_table, edge_proj_w, edge_proj_b,
           g_w, g_b, lg_w, lg_b, r_w1, r_b1, r_w2, r_b2):
    gei = g_edge_index.reshape(-1)
    lgei = lg_edge_index.reshape(-1)
    degp_g, degp_lg, h0 = _prep(gei, lgei, z, emb_table)
    degp_g = degp_g.reshape(2, 2, N_NODES)
    degp_lg = degp_lg.reshape(2, 2, N_EDGES)
    rs_og = jnp.clip(degp_g[0].sum(0), 1.0, None) ** -0.5
    rs_ig = jnp.clip(degp_g[1].sum(0), 1.0, None) ** -0.5
    rs_ol = jnp.clip(degp_lg[0].sum(0), 1.0, None) ** -0.5
    rs_il = jnp.clip(degp_lg[1].sum(0), 1.0, None) ** -0.5

    h = h0
    e = d @ edge_proj_w + edge_proj_b
    g_src, g_dst = g_edge_index[0], g_edge_index[1]
    lg_src, lg_dst = lg_edge_index[0], lg_edge_index[1]
    for i in range(L):
        h = jax.nn.relu(_graph_conv(h, g_src, g_dst, N_NODES, rs_og, rs_ig, g_w[i], g_b[i]))
    for i in range(L):
        e = jax.nn.relu(_graph_conv(e, lg_src, lg_dst, N_EDGES, rs_ol, rs_il, lg_w[i], lg_b[i]))
    hsum = _colsum(h, 2000)
    esum = _colsum(e, 1000)
    out = _readout(hsum, esum, r_w1, r_b1, r_w2, r_b2)
    return out.reshape(-1)


# speedup vs baseline: 1.4466x; 1.1844x over previous
import functools

import jax
import jax.numpy as jnp
from jax import lax
from jax.experimental import pallas as pl
from jax.experimental.pallas import tpu as pltpu
from jax.experimental.pallas import tpu_sc as plsc

N_NODES = 10000
N_EDGES = 320000
LG_EDGES = 640000
H = 128
L = 3

_NC, _NS = 2, 16
_PH = jax.lax.Precision.HIGHEST

_sc_mesh = plsc.VectorSubcoreMesh(core_axis_name="c", subcore_axis_name="s")


def _prep_body(gei, lgei, z, emb, degp_g, degp_lg, h0,
               zbuf, ones, idxb, zidx, rows, obl, ibl, obg, ibg):
    c = lax.axis_index("c")
    s = lax.axis_index("s")
    w = c * _NS + s

    def zf(i, carry):
        zbuf[pl.ds(i * 16, 16)] = jnp.zeros((16,), jnp.float32)
        return carry

    lax.fori_loop(0, 250, zf, 0)
    for i in range(5):
        ones[pl.ds(i * 16, 16)] = jnp.ones((16,), jnp.float32)

    for r in (obl, ibl):
        for k in range(5):
            pltpu.sync_copy(zbuf, r.at[pl.ds(s * 20000 + k * 4000, 4000)])

    @pl.when(s < 10)
    def _zg():
        pltpu.sync_copy(zbuf.at[pl.ds(0, 1000)], obg.at[pl.ds(s * 1000, 1000)])
        pltpu.sync_copy(zbuf.at[pl.ds(0, 1000)], ibg.at[pl.ds(s * 1000, 1000)])

    plsc.subcore_barrier()

    base_lg = c * 320000 + s * 20000

    def dlg(t, carry):
        off = base_lg + t * 80
        pltpu.sync_copy(lgei.at[pl.ds(off, 80)], idxb.at[0])
        pltpu.sync_copy(lgei.at[pl.ds(LG_EDGES + off, 80)], idxb.at[1])
        pltpu.sync_copy(ones, obl.at[idxb.at[0]], add=True)
        pltpu.sync_copy(ones, ibl.at[idxb.at[1]], add=True)
        return carry

    lax.fori_loop(0, 250, dlg, 0)

    base_g = c * 160000 + s * 10000

    def dg(t, carry):
        off = base_g + t * 80
        pltpu.sync_copy(gei.at[pl.ds(off, 80)], idxb.at[0])
        pltpu.sync_copy(gei.at[pl.ds(N_EDGES + off, 80)], idxb.at[1])
        pltpu.sync_copy(ones, obg.at[idxb.at[0]], add=True)
        pltpu.sync_copy(ones, ibg.at[idxb.at[1]], add=True)
        return carry

    lax.fori_loop(0, 125, dg, 0)

    def egather(k, carry):
        t = w + 32 * k

        @pl.when(t < 125)
        def _eg():
            off = t * 80
            pltpu.sync_copy(z.at[pl.ds(off, 80)], zidx.at[0])
            pltpu.sync_copy(emb.at[zidx.at[0]], rows)
            pltpu.sync_copy(rows, h0.at[pl.ds(off, 80)])

        return carry

    lax.fori_loop(0, 4, egather, 0)

    plsc.subcore_barrier()

    for j, r in ((0, obl), (1, ibl)):
        for k in range(5):
            pltpu.sync_copy(r.at[pl.ds(s * 20000 + k * 4000, 4000)], zbuf)
            pltpu.sync_copy(zbuf, degp_lg.at[pl.ds((2 * j + c) * N_EDGES + s * 20000 + k * 4000, 4000)])

    @pl.when(s < 10)
    def _wg():
        for j, r in ((0, obg), (1, ibg)):
            pltpu.sync_copy(r.at[pl.ds(s * 1000, 1000)], zbuf.at[pl.ds(0, 1000)])
            pltpu.sync_copy(zbuf.at[pl.ds(0, 1000)], degp_g.at[pl.ds((2 * j + c) * N_NODES + s * 1000, 1000)])


_prep = pl.kernel(
    _prep_body,
    out_type=(
        jax.ShapeDtypeStruct((4 * N_NODES,), jnp.float32),
        jax.ShapeDtypeStruct((4 * N_EDGES,), jnp.float32),
        jax.ShapeDtypeStruct((N_NODES, H), jnp.float32),
    ),
    mesh=_sc_mesh,
    scratch_types=[
        pltpu.VMEM((4000,), jnp.float32),
        pltpu.VMEM((80,), jnp.float32),
        pltpu.VMEM((2, 80), jnp.int32),
        pltpu.VMEM((1, 80), jnp.int32),
        pltpu.VMEM((80, H), jnp.float32),
        pltpu.VMEM_SHARED((N_EDGES,), jnp.float32),
        pltpu.VMEM_SHARED((N_EDGES,), jnp.float32),
        pltpu.VMEM_SHARED((N_NODES,), jnp.float32),
        pltpu.VMEM_SHARED((N_NODES,), jnp.float32),
    ],
)


def _zero_zb(zb):
    for i in range(16):
        for j in range(8):
            zb[i, pl.ds(j * 16, 16)] = jnp.zeros((16,), jnp.float32)


def _spmm_g_body(hs, gei, aggp, sidx, didx, rows, zb, agg_sp):
    c = lax.axis_index("c")
    s = lax.axis_index("s")
    _zero_zb(zb)

    def zcp(k, carry):
        pltpu.sync_copy(zb, agg_sp.at[pl.ds(s * 624 + k * 16, 16)])
        return carry

    lax.fori_loop(0, 39, zcp, 0)

    @pl.when(s == 15)
    def _ztail():
        pltpu.sync_copy(zb, agg_sp.at[pl.ds(9984, 16)])

    plsc.subcore_barrier()

    base = c * 160000 + s * 10000

    def eloop(t, carry):
        off = base + t * 80
        pltpu.sync_copy(gei.at[pl.ds(off, 80)], sidx.at[0])
        pltpu.sync_copy(hs.at[sidx.at[0]], rows)
        pltpu.sync_copy(gei.at[pl.ds(N_EDGES + off, 80)], didx.at[0])
        pltpu.sync_copy(rows, agg_sp.at[didx.at[0]], add=True)
        return carry

    lax.fori_loop(0, 125, eloop, 0)
    plsc.subcore_barrier()

    def wcp(k, carry):
        pltpu.sync_copy(agg_sp.at[pl.ds(s * 624 + k * 16, 16)], zb)
        pltpu.sync_copy(zb, aggp.at[pl.ds(c * N_NODES + s * 624 + k * 16, 16)])
        return carry

    lax.fori_loop(0, 39, wcp, 0)

    @pl.when(s == 15)
    def _wtail():
        pltpu.sync_copy(agg_sp.at[pl.ds(9984, 16)], zb)
        pltpu.sync_copy(zb, aggp.at[pl.ds(c * N_NODES + 9984, 16)])


_spmm_g = pl.kernel(
    _spmm_g_body,
    out_type=jax.ShapeDtypeStruct((2 * N_NODES, H), jnp.float32),
    mesh=_sc_mesh,
    scratch_types=[
        pltpu.VMEM((1, 80), jnp.int32),
        pltpu.VMEM((1, 80), jnp.int32),
        pltpu.VMEM((80, H), jnp.float32),
        pltpu.VMEM((16, H), jnp.float32),
        pltpu.VMEM_SHARED((N_NODES, H), jnp.float32),
    ],
)


def _colsum_body(x_ref, o_ref):
    @pl.when(pl.program_id(0) == 0)
    def _init():
        o_ref[...] = jnp.zeros_like(o_ref)

    o_ref[...] += jnp.sum(x_ref[...], axis=0, keepdims=True)


def _colsum(x, block_rows):
    n = x.shape[0]
    return pl.pallas_call(
        _colsum_body,
        grid=(n // block_rows,),
        in_specs=[pl.BlockSpec((block_rows, H), lambda i: (i, 0))],
        out_specs=pl.BlockSpec((1, H), lambda i: (0, 0)),
        out_shape=jax.ShapeDtypeStruct((1, H), jnp.float32),
    )(x)


def _readout_body(hs_ref, es_ref, w1_ref, b1_ref, w2_ref, b2_ref, o_ref):
    x = jnp.concatenate([hs_ref[...] / N_NODES, es_ref[...] / N_EDGES], axis=-1)
    y = jnp.dot(x, w1_ref[...], precision=_PH) + b1_ref[...]
    y = y * jax.nn.sigmoid(y)
    o_ref[...] = jnp.dot(y, w2_ref[...], precision=_PH) + b2_ref[...]


def _readout(hsum, esum, w1, b1, w2, b2):
    return pl.pallas_call(
        _readout_body,
        out_shape=jax.ShapeDtypeStruct((1, 1), jnp.float32),
    )(hsum, esum, w1, b1.reshape(1, H), w2, b2.reshape(1, 1))


def _graph_conv(h, src, dst, n, rs_out, rs_in, W, b):
    h = h * rs_out[:, None]
    msg = jnp.take(h, src, axis=0)
    agg = jnp.zeros((n, h.shape[1]), h.dtype).at[dst].add(msg)
    agg = agg * rs_in[:, None]
    return agg @ W + b


def kernel(z, g_edge_index, d, lg_edge_index, emb_table, edge_proj_w, edge_proj_b,
           g_w, g_b, lg_w, lg_b, r_w1, r_b1, r_w2, r_b2):
    gei = g_edge_index.reshape(-1)
    lgei = lg_edge_index.reshape(-1)
    degp_g, degp_lg, h0 = _prep(gei, lgei, z, emb_table)
    degp_g = degp_g.reshape(2, 2, N_NODES)
    degp_lg = degp_lg.reshape(2, 2, N_EDGES)
    rs_og = jnp.clip(degp_g[0].sum(0), 1.0, None) ** -0.5
    rs_ig = jnp.clip(degp_g[1].sum(0), 1.0, None) ** -0.5
    rs_ol = jnp.clip(degp_lg[0].sum(0), 1.0, None) ** -0.5
    rs_il = jnp.clip(degp_lg[1].sum(0), 1.0, None) ** -0.5

    h = h0
    e = d @ edge_proj_w + edge_proj_b
    lg_src, lg_dst = lg_edge_index[0], lg_edge_index[1]
    for i in range(L):
        hs = h * rs_og[:, None]
        aggp = _spmm_g(hs, gei)
        agg = (aggp[:N_NODES] + aggp[N_NODES:]) * rs_ig[:, None]
        h = jax.nn.relu(agg @ g_w[i] + g_b[i])
    for i in range(L):
        e = jax.nn.relu(_graph_conv(e, lg_src, lg_dst, N_EDGES, rs_ol, rs_il, lg_w[i], lg_b[i]))
    hsum = _colsum(h, 2000)
    esum = _colsum(e, 1000)
    out = _readout(hsum, esum, r_w1, r_b1, r_w2, r_b2)
    return out.reshape(-1)
